# Initial kernel scaffold; baseline (speedup 1.0000x reference)
import jax
import jax.numpy as jnp
from jax.experimental import pallas as pl


def kernel(x, Wq, bq, Wk, bk, Wv, bv, Wo, bo, g1, beta1, g2, beta2, W1, bf1, W2, bf2, rel_emb):
    raise NotImplementedError("write your pallas kernel here")


# speedup vs baseline: 9.5592x; 9.5592x over previous
import jax
import jax.numpy as jnp
from jax.experimental import pallas as pl
from jax.experimental.pallas import tpu as pltpu

_H = 16
_HD = 64
_TOPK = 32
_RB = 256
_BIASW = 2304

_HIGH = jax.lax.Precision.HIGHEST


def _ln(x, g, b, eps=1e-5):
    mu = jnp.mean(x, axis=1, keepdims=True)
    xc = x - mu
    var = jnp.mean(xc * xc, axis=1, keepdims=True)
    return xc * jax.lax.rsqrt(var + eps) * g + b


def _qkv_kernel(x_ref, w_ref, b_ref, g_ref, beta_ref, o_ref):
    xn = _ln(x_ref[...], g_ref[...], beta_ref[...])
    o_ref[...] = jax.lax.dot_general(
        xn, w_ref[...], (((1,), (0,)), ((), ())),
        precision=_HIGH, preferred_element_type=jnp.float32) + b_ref[...]


def _attn_kernel(q_ref, kt_ref, v_ref, relw_ref, o_ref):
    n = kt_ref.shape[2]
    scale = _HD ** -0.5
    q = q_ref[0]
    kt = kt_ref[0]
    logits = jax.lax.dot_general(
        q, kt, (((1,), (0,)), ((), ())),
        precision=_HIGH, preferred_element_type=jnp.float32) * scale

    m = jnp.broadcast_to(relw_ref[0], (_RB, _BIASW))
    r = jax.lax.broadcasted_iota(jnp.int32, (_RB, _BIASW), 0)
    for k in range(8):
        sh = 1 << k
        mrot = jnp.roll(m, -sh, axis=1)
        m = jnp.where(((r >> k) & 1) == 0, mrot, m)
    logits = logits + m[:, :n]

    rowmax = jnp.max(logits, axis=1, keepdims=True)

    def body(_, carry):
        work, _m = carry
        cur = jnp.max(work, axis=1, keepdims=True)
        work = jnp.where(work == cur, -jnp.inf, work)
        return (work, cur)

    _, thresh = jax.lax.fori_loop(0, _TOPK, body, (logits, rowmax))

    p = jnp.where(logits >= thresh, jnp.exp(logits - rowmax), 0.0)
    denom = jnp.sum(p, axis=1, keepdims=True)
    pv = jax.lax.dot_general(
        p, v_ref[0], (((1,), (0,)), ((), ())),
        precision=_HIGH, preferred_element_type=jnp.float32)
    o_ref[0] = pv / denom


def _ffn_kernel(x_ref, ao_ref, wo_ref, bo_ref, g2_ref, b2_ref,
                w1_ref, bf1_ref, w2_ref, bf2_ref, o_ref):
    x = x_ref[...]
    proj = jax.lax.dot_general(
        ao_ref[...].astype(jnp.bfloat16), wo_ref[...], (((1,), (0,)), ((), ())),
        preferred_element_type=jnp.float32)
    x1 = x + proj + bo_ref[...]
    xn2 = _ln(x1, g2_ref[...], b2_ref[...])
    h = jax.lax.dot_general(
        xn2.astype(jnp.bfloat16), w1_ref[...], (((1,), (0,)), ((), ())),
        preferred_element_type=jnp.float32) + bf1_ref[...]
    h = 0.5 * h * (1.0 + jax.lax.erf(h * 0.7071067811865476))
    ff = jax.lax.dot_general(
        h.astype(jnp.bfloat16), w2_ref[...], (((1,), (0,)), ((), ())),
        preferred_element_type=jnp.float32) + bf2_ref[...]
    o_ref[...] = x1 + ff


def kernel(x, Wq, bq, Wk, bk, Wv, bv, Wo, bo, g1, beta1, g2, beta2,
           W1, bf1, W2, bf2, rel_emb):
    b, n, d = x.shape
    nb = n // _RB
    x2 = x.reshape(n, d)

    wqkv = jnp.concatenate([Wq.T, Wk.T, Wv.T], axis=1)
    bqkv = jnp.concatenate([bq, bk, bv]).reshape(1, 3 * d)
    y = pl.pallas_call(
        _qkv_kernel,
        grid=(nb,),
        in_specs=[
            pl.BlockSpec((_RB, d), lambda i: (i, 0)),
            pl.BlockSpec((d, 3 * d), lambda i: (0, 0)),
            pl.BlockSpec((1, 3 * d), lambda i: (0, 0)),
            pl.BlockSpec((1, d), lambda i: (0, 0)),
            pl.BlockSpec((1, d), lambda i: (0, 0)),
        ],
        out_specs=pl.BlockSpec((_RB, 3 * d), lambda i: (i, 0)),
        out_shape=jax.ShapeDtypeStruct((n, 3 * d), jnp.float32),
    )(x2, wqkv, bqkv, g1.reshape(1, d), beta1.reshape(1, d))

    q, kk, v = jnp.split(y, 3, axis=1)
    qh = q.reshape(n, _H, _HD).transpose(1, 0, 2)
    kth = kk.reshape(n, _H, _HD).transpose(1, 2, 0)
    vh = v.reshape(n, _H, _HD).transpose(1, 0, 2)

    maxseq = (rel_emb.shape[0] + 1) // 2
    relt = jnp.pad(rel_emb.T, ((0, 0), (0, 1)))
    starts = [maxseq - _RB - bi * _RB for bi in range(nb)]
    relw = jnp.stack(
        [relt[:, s:s + _BIASW] for s in starts], axis=1)
    relw = relw.reshape(_H * nb, 1, _BIASW)

    ao = pl.pallas_call(
        _attn_kernel,
        grid=(_H, nb),
        in_specs=[
            pl.BlockSpec((1, _RB, _HD), lambda h, bi: (h, bi, 0)),
            pl.BlockSpec((1, _HD, n), lambda h, bi: (h, 0, 0)),
            pl.BlockSpec((1, n, _HD), lambda h, bi: (h, 0, 0)),
            pl.BlockSpec((1, 1, _BIASW), lambda h, bi, nb=nb: (h * nb + bi, 0, 0)),
        ],
        out_specs=pl.BlockSpec((1, _RB, _HD), lambda h, bi: (h, bi, 0)),
        out_shape=jax.ShapeDtypeStruct((_H, n, _HD), jnp.float32),
    )(qh, kth, vh, relw)
    ao2 = ao.transpose(1, 0, 2).reshape(n, d)

    out = pl.pallas_call(
        _ffn_kernel,
        grid=(nb,),
        in_specs=[
            pl.BlockSpec((_RB, d), lambda i: (i, 0)),
            pl.BlockSpec((_RB, d), lambda i: (i, 0)),
            pl.BlockSpec((d, d), lambda i: (0, 0)),
            pl.BlockSpec((1, d), lambda i: (0, 0)),
            pl.BlockSpec((1, d), lambda i: (0, 0)),
            pl.BlockSpec((1, d), lambda i: (0, 0)),
            pl.BlockSpec((d, 4 * d), lambda i: (0, 0)),
            pl.BlockSpec((1, 4 * d), lambda i: (0, 0)),
            pl.BlockSpec((4 * d, d), lambda i: (0, 0)),
            pl.BlockSpec((1, d), lambda i: (0, 0)),
        ],
        out_specs=pl.BlockSpec((_RB, d), lambda i: (i, 0)),
        out_shape=jax.ShapeDtypeStruct((n, d), jnp.float32),
    )(x2, ao2, Wo.T.astype(jnp.bfloat16), bo.reshape(1, d),
      g2.reshape(1, d), beta2.reshape(1, d),
      W1.T.astype(jnp.bfloat16), bf1.reshape(1, 4 * d),
      W2.T.astype(jnp.bfloat16), bf2.reshape(1, d))

    return out.reshape(b, n, d)


# speedup vs baseline: 11.5101x; 1.2041x over previous
import jax
import jax.numpy as jnp
from jax.experimental import pallas as pl
from jax.experimental.pallas import tpu as pltpu

_H = 16
_HD = 64
_TOPK = 32
_RB = 256
_BIASW = 2304

_HIGH = jax.lax.Precision.HIGHEST


def _ln(x, g, b, eps=1e-5):
    mu = jnp.mean(x, axis=1, keepdims=True)
    xc = x - mu
    var = jnp.mean(xc * xc, axis=1, keepdims=True)
    return xc * jax.lax.rsqrt(var + eps) * g + b


def _qkv_kernel(x_ref, w_ref, b_ref, g_ref, beta_ref, o_ref):
    xn = _ln(x_ref[...], g_ref[...], beta_ref[...])
    o_ref[...] = jax.lax.dot_general(
        xn, w_ref[...], (((1,), (0,)), ((), ())),
        precision=_HIGH, preferred_element_type=jnp.float32) + b_ref[...]


def _attn_kernel(q_ref, kt_ref, v_ref, relw_ref, o_ref):
    n = kt_ref.shape[2]
    scale = _HD ** -0.5
    q = q_ref[0]
    kt = kt_ref[0]
    logits = jax.lax.dot_general(
        q, kt, (((1,), (0,)), ((), ())),
        precision=_HIGH, preferred_element_type=jnp.float32) * scale

    m = jnp.broadcast_to(relw_ref[0], (_RB, _BIASW))
    r = jax.lax.broadcasted_iota(jnp.int32, (_RB, _BIASW), 0)
    for k in range(8):
        sh = 1 << k
        mrot = jnp.roll(m, -sh, axis=1)
        m = jnp.where(((r >> k) & 1) == 0, mrot, m)
    logits = logits + m[:, :n]

    c = n // 4
    a0, a1, a2, a3 = (logits[:, :c], logits[:, c:2 * c],
                      logits[:, 2 * c:3 * c], logits[:, 3 * c:])

    def _ce(x, y):
        return jnp.maximum(x, y), jnp.minimum(x, y)

    a0, a2 = _ce(a0, a2)
    a1, a3 = _ce(a1, a3)
    a0, a1 = _ce(a0, a1)
    a2, a3 = _ce(a2, a3)
    a1, a2 = _ce(a1, a2)

    rowmax = jnp.max(a0, axis=1, keepdims=True)
    neg = jnp.float32(-jnp.inf)

    def body(_, carry):
        s, l1, l2, l3, _m = carry
        cur = jnp.max(s, axis=1, keepdims=True)
        hit = s == cur
        s = jnp.where(hit, l1, s)
        l1 = jnp.where(hit, l2, l1)
        l2 = jnp.where(hit, l3, l2)
        l3 = jnp.where(hit, neg, l3)
        return (s, l1, l2, l3, cur)

    _, _, _, _, thresh = jax.lax.fori_loop(
        0, _TOPK, body, (a0, a1, a2, a3, rowmax))

    p = jnp.where(logits >= thresh, jnp.exp(logits - rowmax), 0.0)
    denom = jnp.sum(p, axis=1, keepdims=True)
    pv = jax.lax.dot_general(
        p.astype(jnp.bfloat16), v_ref[0], (((1,), (0,)), ((), ())),
        preferred_element_type=jnp.float32)
    o_ref[0] = pv / denom


def _ffn_kernel(x_ref, ao_ref, wo_ref, bo_ref, g2_ref, b2_ref,
                w1_ref, bf1_ref, w2_ref, bf2_ref, o_ref):
    x = x_ref[...]
    proj = jax.lax.dot_general(
        ao_ref[...].astype(jnp.bfloat16), wo_ref[...], (((1,), (0,)), ((), ())),
        preferred_element_type=jnp.float32)
    x1 = x + proj + bo_ref[...]
    xn2 = _ln(x1, g2_ref[...], b2_ref[...])
    h = jax.lax.dot_general(
        xn2.astype(jnp.bfloat16), w1_ref[...], (((1,), (0,)), ((), ())),
        preferred_element_type=jnp.float32) + bf1_ref[...]
    h = 0.5 * h * (1.0 + jax.lax.erf(h * 0.7071067811865476))
    ff = jax.lax.dot_general(
        h.astype(jnp.bfloat16), w2_ref[...], (((1,), (0,)), ((), ())),
        preferred_element_type=jnp.float32) + bf2_ref[...]
    o_ref[...] = x1 + ff


def kernel(x, Wq, bq, Wk, bk, Wv, bv, Wo, bo, g1, beta1, g2, beta2,
           W1, bf1, W2, bf2, rel_emb):
    b, n, d = x.shape
    nb = n // _RB
    x2 = x.reshape(n, d)

    wqkv = jnp.concatenate([Wq.T, Wk.T, Wv.T], axis=1)
    bqkv = jnp.concatenate([bq, bk, bv]).reshape(1, 3 * d)
    y = pl.pallas_call(
        _qkv_kernel,
        grid=(nb,),
        in_specs=[
            pl.BlockSpec((_RB, d), lambda i: (i, 0)),
            pl.BlockSpec((d, 3 * d), lambda i: (0, 0)),
            pl.BlockSpec((1, 3 * d), lambda i: (0, 0)),
            pl.BlockSpec((1, d), lambda i: (0, 0)),
            pl.BlockSpec((1, d), lambda i: (0, 0)),
        ],
        out_specs=pl.BlockSpec((_RB, 3 * d), lambda i: (i, 0)),
        out_shape=jax.ShapeDtypeStruct((n, 3 * d), jnp.float32),
    )(x2, wqkv, bqkv, g1.reshape(1, d), beta1.reshape(1, d))

    q, kk, v = jnp.split(y, 3, axis=1)
    qh = q.reshape(n, _H, _HD).transpose(1, 0, 2)
    kth = kk.reshape(n, _H, _HD).transpose(1, 2, 0)
    vh = v.reshape(n, _H, _HD).transpose(1, 0, 2).astype(jnp.bfloat16)

    maxseq = (rel_emb.shape[0] + 1) // 2
    relt = jnp.pad(rel_emb.T, ((0, 0), (0, 1)))
    starts = [maxseq - _RB - bi * _RB for bi in range(nb)]
    relw = jnp.stack(
        [relt[:, s:s + _BIASW] for s in starts], axis=1)
    relw = relw.reshape(_H * nb, 1, _BIASW)

    ao = pl.pallas_call(
        _attn_kernel,
        grid=(_H, nb),
        in_specs=[
            pl.BlockSpec((1, _RB, _HD), lambda h, bi: (h, bi, 0)),
            pl.BlockSpec((1, _HD, n), lambda h, bi: (h, 0, 0)),
            pl.BlockSpec((1, n, _HD), lambda h, bi: (h, 0, 0)),
            pl.BlockSpec((1, 1, _BIASW), lambda h, bi, nb=nb: (h * nb + bi, 0, 0)),
        ],
        out_specs=pl.BlockSpec((1, _RB, _HD), lambda h, bi: (h, bi, 0)),
        out_shape=jax.ShapeDtypeStruct((_H, n, _HD), jnp.float32),
    )(qh, kth, vh, relw)
    ao2 = ao.transpose(1, 0, 2).reshape(n, d)

    out = pl.pallas_call(
        _ffn_kernel,
        grid=(nb,),
        in_specs=[
            pl.BlockSpec((_RB, d), lambda i: (i, 0)),
            pl.BlockSpec((_RB, d), lambda i: (i, 0)),
            pl.BlockSpec((d, d), lambda i: (0, 0)),
            pl.BlockSpec((1, d), lambda i: (0, 0)),
            pl.BlockSpec((1, d), lambda i: (0, 0)),
            pl.BlockSpec((1, d), lambda i: (0, 0)),
            pl.BlockSpec((d, 4 * d), lambda i: (0, 0)),
            pl.BlockSpec((1, 4 * d), lambda i: (0, 0)),
            pl.BlockSpec((4 * d, d), lambda i: (0, 0)),
            pl.BlockSpec((1, d), lambda i: (0, 0)),
        ],
        out_specs=pl.BlockSpec((_RB, d), lambda i: (i, 0)),
        out_shape=jax.ShapeDtypeStruct((n, d), jnp.float32),
    )(x2, ao2, Wo.T.astype(jnp.bfloat16), bo.reshape(1, d),
      g2.reshape(1, d), beta2.reshape(1, d),
      W1.T.astype(jnp.bfloat16), bf1.reshape(1, 4 * d),
      W2.T.astype(jnp.bfloat16), bf2.reshape(1, d))

    return out.reshape(b, n, d)
